# baseline (speedup 1.0000x reference)
import functools

import jax
import jax.numpy as jnp
from jax import lax
from jax.experimental import pallas as pl
from jax.experimental.pallas import tpu as pltpu
from jax.experimental.pallas import tpu_sc as plsc

D_MODEL = 768
N_PATCHES = 576
N_TOT = N_PATCHES + 1
BATCH = 128

NC = 2
NS = 16
NW = NC * NS
ROWS = 18
LANES = 16
VECS_PER_ROW = D_MODEL // LANES


def _sc_body(in_hbm, cls_hbm, pos_hbm, out_hbm, inbuf, outbuf, posbuf, clsbuf):
  wid = lax.axis_index("c") * NS + lax.axis_index("s")
  is_w0 = wid == 0
  off = jnp.where(is_w0, 1, 0)

  @pl.when(is_w0)
  def _():
    pltpu.sync_copy(pos_hbm.at[pl.ds(0, ROWS + 1)], posbuf)
    pltpu.sync_copy(cls_hbm, clsbuf)

  @pl.when(jnp.logical_not(is_w0))
  def _():
    pltpu.sync_copy(pos_hbm.at[pl.ds(ROWS * wid + 1, ROWS)],
                    posbuf.at[pl.ds(0, ROWS)])

  @pl.when(is_w0)
  def _():
    for c in range(VECS_PER_ROW):
      sl = pl.ds(c * LANES, LANES)
      outbuf[0, sl] = clsbuf[sl] + posbuf[0, sl]

  def batch_body(b, carry):
    pltpu.sync_copy(in_hbm.at[pl.ds(b * N_PATCHES + ROWS * wid, ROWS)], inbuf)

    def row_body(j, carry2):
      r = off + j
      for c in range(VECS_PER_ROW):
        sl = pl.ds(c * LANES, LANES)
        outbuf[r, sl] = inbuf[j, sl] + posbuf[r, sl]
      return carry2

    lax.fori_loop(0, ROWS, row_body, 0)

    out_row = b * N_TOT + ROWS * wid

    @pl.when(is_w0)
    def _():
      pltpu.sync_copy(outbuf, out_hbm.at[pl.ds(out_row, ROWS + 1)])

    @pl.when(jnp.logical_not(is_w0))
    def _():
      pltpu.sync_copy(outbuf.at[pl.ds(0, ROWS)],
                      out_hbm.at[pl.ds(out_row + 1, ROWS)])

    return carry

  lax.fori_loop(0, BATCH, batch_body, 0)


@jax.jit
def kernel(inputs, class_embed, pos_table):
  in2d = inputs.reshape(BATCH * N_PATCHES, D_MODEL)
  cls1d = class_embed.reshape(D_MODEL)
  mesh = plsc.VectorSubcoreMesh(core_axis_name="c", subcore_axis_name="s")
  run = functools.partial(
      pl.kernel,
      mesh=mesh,
      compiler_params=pltpu.CompilerParams(use_tc_tiling_on_sc=False),
      out_type=jax.ShapeDtypeStruct((BATCH * N_TOT, D_MODEL), jnp.float32),
      scratch_types=[
          pltpu.VMEM((ROWS, D_MODEL), jnp.float32),
          pltpu.VMEM((ROWS + 1, D_MODEL), jnp.float32),
          pltpu.VMEM((ROWS + 1, D_MODEL), jnp.float32),
          pltpu.VMEM((D_MODEL,), jnp.float32),
      ],
  )(_sc_body)
  out2d = run(in2d, cls1d, pos_table)
  return out2d.reshape(BATCH, N_TOT, D_MODEL)


# speedup vs baseline: 1.7954x; 1.7954x over previous
import functools

import jax
import jax.numpy as jnp
from jax import lax
from jax.experimental import pallas as pl
from jax.experimental.pallas import tpu as pltpu
from jax.experimental.pallas import tpu_sc as plsc

D_MODEL = 768
N_PATCHES = 576
N_TOT = N_PATCHES + 1
BATCH = 128

NC = 2
NS = 16
NW = NC * NS
ROWS = N_PATCHES // NW
CHUNK = ROWS * D_MODEL
LANES = 16
UNROLL = 8
N_VECS = CHUNK // LANES
N_STEPS = N_VECS // UNROLL


def _sc_body(in_hbm, cls_hbm, pos_hbm, out_hbm,
             inbuf, outbuf, posbuf, clsrow, pos0,
             in_sem0, in_sem1, out_sem0, out_sem1, cls_sem):
  wid = lax.axis_index("c") * NS + lax.axis_index("s")
  is_w0 = wid == 0
  in_sems = (in_sem0, in_sem1)
  out_sems = (out_sem0, out_sem1)

  pltpu.sync_copy(pos_hbm.at[pl.ds((ROWS * wid + 1) * D_MODEL, CHUNK)], posbuf)

  @pl.when(is_w0)
  def _():
    pltpu.sync_copy(cls_hbm, clsrow)
    pltpu.sync_copy(pos_hbm.at[pl.ds(0, D_MODEL)], pos0)
    for c in range(D_MODEL // LANES):
      sl = pl.ds(c * LANES, LANES)
      clsrow[sl] = clsrow[sl] + pos0[sl]

  def in_copy(b, s):
    return pltpu.make_async_copy(
        in_hbm.at[pl.ds((b * N_PATCHES + ROWS * wid) * D_MODEL, CHUNK)],
        inbuf.at[s], in_sems[s])

  def out_copy(b, s):
    return pltpu.make_async_copy(
        outbuf.at[s],
        out_hbm.at[pl.ds((b * N_TOT + ROWS * wid + 1) * D_MODEL, CHUNK)],
        out_sems[s])

  def cls_copy(b):
    return pltpu.make_async_copy(
        clsrow, out_hbm.at[pl.ds(b * N_TOT * D_MODEL, D_MODEL)], cls_sem)

  in_copy(0, 0).start()
  in_copy(1, 1).start()

  def step(b, s, not_first, not_last):
    in_copy(b, s).wait()

    @pl.when(not_first)
    def _():
      out_copy(b - 2, s).wait()

    @pl.when(is_w0)
    def _():
      @pl.when(not_first)
      def _():
        cls_copy(b - 2).wait()
      cls_copy(b).start()

    def add_block(k, carry):
      base = k * (UNROLL * LANES)
      for u in range(UNROLL):
        sl = pl.ds(base + u * LANES, LANES)
        outbuf[s, sl] = inbuf[s, sl] + posbuf[sl]
      return carry

    lax.fori_loop(0, N_STEPS, add_block, 0)

    out_copy(b, s).start()

    @pl.when(not_last)
    def _():
      in_copy(b + 2, s).start()

  def pair(g, carry):
    not_first = g >= 1
    not_last = g < (BATCH // 2 - 1)
    step(2 * g, 0, not_first, not_last)
    step(2 * g + 1, 1, not_first, not_last)
    return carry

  lax.fori_loop(0, BATCH // 2, pair, 0)

  out_copy(BATCH - 2, 0).wait()
  out_copy(BATCH - 1, 1).wait()

  @pl.when(is_w0)
  def _():
    cls_copy(BATCH - 2).wait()
    cls_copy(BATCH - 1).wait()


@jax.jit
def kernel(inputs, class_embed, pos_table):
  in1d = inputs.reshape(BATCH * N_PATCHES * D_MODEL)
  cls1d = class_embed.reshape(D_MODEL)
  pos1d = pos_table.reshape(N_TOT * D_MODEL)
  mesh = plsc.VectorSubcoreMesh(core_axis_name="c", subcore_axis_name="s")
  run = functools.partial(
      pl.kernel,
      mesh=mesh,
      compiler_params=pltpu.CompilerParams(use_tc_tiling_on_sc=False),
      out_type=jax.ShapeDtypeStruct((BATCH * N_TOT * D_MODEL,), jnp.float32),
      scratch_types=[
          pltpu.VMEM((2, CHUNK), jnp.float32),
          pltpu.VMEM((2, CHUNK), jnp.float32),
          pltpu.VMEM((CHUNK,), jnp.float32),
          pltpu.VMEM((D_MODEL,), jnp.float32),
          pltpu.VMEM((D_MODEL,), jnp.float32),
          pltpu.SemaphoreType.DMA,
          pltpu.SemaphoreType.DMA,
          pltpu.SemaphoreType.DMA,
          pltpu.SemaphoreType.DMA,
          pltpu.SemaphoreType.DMA,
      ],
  )(_sc_body)
  out1d = run(in1d, cls1d, pos1d)
  return out1d.reshape(BATCH, N_TOT, D_MODEL)
